# baseline (speedup 1.0000x reference)
import jax
import jax.numpy as jnp
from jax.experimental import pallas as pl
from functools import partial

B, C_IN, H, W = 16, 1, 224, 224
HID = 128
K = 1024
D = 64
N_TOK = 16 * 56 * 56
TILE = 1024
GRID = N_TOK // TILE


def _vq_body(flat_ref, emb_ref, embt_ref, embsq_ref,
             quant_ref, counts_ref, sqerr_ref):
    i = pl.program_id(0)
    f = flat_ref[...]
    emb = emb_ref[...]
    scores = jax.lax.dot_general(
        f, emb, (((1,), (0,)), ((), ())),
        preferred_element_type=jnp.float32)
    row_sq = jnp.sum(f * f, axis=1, keepdims=True)
    d = (row_sq + embsq_ref[...]) - 2.0 * scores
    m = jnp.min(d, axis=1, keepdims=True)
    iota = jax.lax.broadcasted_iota(jnp.int32, (TILE, K), 1)
    idx = jnp.min(jnp.where(d == m, iota, K), axis=1)
    onehot = (iota == idx[:, None]).astype(jnp.float32)
    q = jax.lax.dot_general(
        onehot, embt_ref[...], (((1,), (0,)), ((), ())),
        preferred_element_type=jnp.float32,
        precision=jax.lax.Precision.HIGHEST)
    quant_ref[...] = q
    dsq = jnp.sum((q - f) ** 2)
    cnt = jnp.sum(onehot, axis=0, keepdims=True)

    @pl.when(i == 0)
    def _init():
        counts_ref[...] = cnt
        sqerr_ref[...] = jnp.full((1, 128), dsq, jnp.float32)

    @pl.when(i != 0)
    def _acc():
        counts_ref[...] += cnt
        sqerr_ref[...] += dsq


def _vq(flat, embeddings):
    emb_sq = jnp.sum(embeddings * embeddings, axis=0, keepdims=True)
    quant, counts, sqerr = pl.pallas_call(
        _vq_body,
        grid=(GRID,),
        in_specs=[
            pl.BlockSpec((TILE, D), lambda i: (i, 0)),
            pl.BlockSpec((D, K), lambda i: (0, 0)),
            pl.BlockSpec((K, D), lambda i: (0, 0)),
            pl.BlockSpec((1, K), lambda i: (0, 0)),
        ],
        out_specs=[
            pl.BlockSpec((TILE, D), lambda i: (i, 0)),
            pl.BlockSpec((1, K), lambda i: (0, 0)),
            pl.BlockSpec((1, 128), lambda i: (0, 0)),
        ],
        out_shape=[
            jax.ShapeDtypeStruct((N_TOK, D), jnp.float32),
            jax.ShapeDtypeStruct((1, K), jnp.float32),
            jax.ShapeDtypeStruct((1, 128), jnp.float32),
        ],
    )(flat, embeddings, embeddings.T, emb_sq)
    return quant, counts[0], sqerr[0, 0]


def _conv2d(x, w, b, stride, pad):
    y = jax.lax.conv_general_dilated(
        x, w, (stride, stride), ((pad, pad), (pad, pad)),
        dimension_numbers=('NCHW', 'OIHW', 'NCHW'))
    return y + b[None, :, None, None]


def _conv_transpose2d(x, w, b, stride, pad, out_pad):
    k = w.shape[2]
    w_t = jnp.transpose(jnp.flip(w, (2, 3)), (1, 0, 2, 3))
    lo = k - 1 - pad
    hi = k - 1 - pad + out_pad
    y = jax.lax.conv_general_dilated(
        x, w_t, (1, 1), ((lo, hi), (lo, hi)), lhs_dilation=(stride, stride),
        dimension_numbers=('NCHW', 'OIHW', 'NCHW'))
    return y + b[None, :, None, None]


@jax.jit
def kernel(x, enc_w1, enc_b1, enc_w2, enc_b2, enc_w3, enc_b3, embeddings,
           dec_w1, dec_b1, dec_w2, dec_b2, dec_w3, dec_b3):
    h = jax.nn.relu(_conv2d(x, enc_w1, enc_b1, 2, 1))
    h = jax.nn.relu(_conv2d(h, enc_w2, enc_b2, 2, 1))
    z = _conv2d(h, enc_w3, enc_b3, 1, 1)
    z_e = jnp.transpose(z, (0, 2, 3, 1))
    flat = z_e.reshape(-1, D)

    quant_flat, counts, sqerr = _vq(flat, embeddings)

    loss = 1.25 * sqerr / (N_TOK * D)
    avg_probs = counts / N_TOK
    perplexity = jnp.exp(-jnp.sum(avg_probs * jnp.log(avg_probs + 1e-10)))

    quantized = quant_flat.reshape(z_e.shape)
    q = jnp.transpose(quantized, (0, 3, 1, 2))
    h = jax.nn.relu(_conv_transpose2d(q, dec_w1, dec_b1, 2, 1, 1))
    h = jax.nn.relu(_conv_transpose2d(h, dec_w2, dec_b2, 2, 1, 1))
    x_recon = _conv_transpose2d(h, dec_w3, dec_b3, 1, 1, 0)
    return (loss, x_recon, perplexity)


# speedup vs baseline: 1.6074x; 1.6074x over previous
import jax
import jax.numpy as jnp
from jax.experimental import pallas as pl

B, C_IN, H, W = 16, 1, 224, 224
HID = 128
K = 1024
D = 64
N_TOK = 16 * 56 * 56
TILE = 1024
GRID = N_TOK // TILE

_S = {0: ((0, 1),), 1: ((0, 2), (1, 0))}


def _vq_body(flat_ref, emb_ref, embt_ref, embsq_ref,
             quant_ref, counts_ref, sqerr_ref):
    i = pl.program_id(0)
    f = flat_ref[...]
    emb = emb_ref[...]
    scores = jax.lax.dot_general(
        f, emb, (((1,), (0,)), ((), ())),
        preferred_element_type=jnp.float32)
    row_sq = jnp.sum(f * f, axis=1, keepdims=True)
    d = (row_sq + embsq_ref[...]) - 2.0 * scores
    m = jnp.min(d, axis=1, keepdims=True)
    iota = jax.lax.broadcasted_iota(jnp.int32, (TILE, K), 1)
    idx = jnp.min(jnp.where(d == m, iota, K), axis=1)
    onehot = (iota == idx[:, None]).astype(jnp.float32)
    q = jax.lax.dot_general(
        onehot, embt_ref[...], (((1,), (0,)), ((), ())),
        preferred_element_type=jnp.float32,
        precision=jax.lax.Precision.HIGHEST)
    quant_ref[...] = q
    dsq = jnp.sum((q - f) ** 2)
    cnt = jnp.sum(onehot, axis=0, keepdims=True)

    @pl.when(i == 0)
    def _init():
        counts_ref[...] = cnt
        sqerr_ref[...] = jnp.full((1, 128), dsq, jnp.float32)

    @pl.when(i != 0)
    def _acc():
        counts_ref[...] += cnt
        sqerr_ref[...] += dsq


def _vq(flat, embeddings):
    emb_sq = jnp.sum(embeddings * embeddings, axis=0, keepdims=True)
    quant, counts, sqerr = pl.pallas_call(
        _vq_body,
        grid=(GRID,),
        in_specs=[
            pl.BlockSpec((TILE, D), lambda i: (i, 0)),
            pl.BlockSpec((D, K), lambda i: (0, 0)),
            pl.BlockSpec((K, D), lambda i: (0, 0)),
            pl.BlockSpec((1, K), lambda i: (0, 0)),
        ],
        out_specs=[
            pl.BlockSpec((TILE, D), lambda i: (i, 0)),
            pl.BlockSpec((1, K), lambda i: (0, 0)),
            pl.BlockSpec((1, 128), lambda i: (0, 0)),
        ],
        out_shape=[
            jax.ShapeDtypeStruct((N_TOK, D), jnp.float32),
            jax.ShapeDtypeStruct((1, K), jnp.float32),
            jax.ShapeDtypeStruct((1, 128), jnp.float32),
        ],
    )(flat, embeddings, embeddings.T, emb_sq)
    return quant, counts[0], sqerr[0, 0]


def _phase_w(w):
    c_in, c_out = w.shape[0], w.shape[1]
    mats = []
    for r in (0, 1):
        for c in (0, 1):
            m = jnp.zeros((4, c_in, c_out), w.dtype)
            for (di, ki) in _S[r]:
                for (dj, kj) in _S[c]:
                    m = m.at[2 * di + dj].set(w[:, :, ki, kj])
            mats.append(m.reshape(4 * c_in, c_out))
    return jnp.stack(mats)


def _shift_rows(a):
    return jnp.concatenate(
        [a[1:], jnp.zeros((1,) + a.shape[1:], a.dtype)], axis=0)


def _shift_cols(a):
    return jnp.concatenate(
        [a[:, 1:], jnp.zeros((a.shape[0], 1) + a.shape[2:], a.dtype)], axis=1)


def _dec1_body(x_ref, w_ref, b_ref, out_ref):
    xb = x_ref[0]
    x_c = _shift_cols(xb)
    x_r = _shift_rows(xb)
    x_rc = _shift_cols(x_r)
    p = jnp.concatenate([xb, x_c, x_r, x_rc], axis=-1).reshape(3136, 4 * D)
    for q in range(4):
        y = jnp.dot(p, w_ref[q], preferred_element_type=jnp.float32)
        y = jnp.maximum(y + b_ref[...], 0.0)
        out_ref[0, :, :, q * HID:(q + 1) * HID] = y.reshape(56, 56, HID)


def _dec2_body(y1_ref, w_ref, b_ref, out_ref):
    y1 = y1_ref[0]
    phases = [y1[:, :, q * HID:(q + 1) * HID] for q in range(4)]
    for r in (0, 1):
        for c in (0, 1):
            comps = []
            for di in (0, 1):
                for dj in (0, 1):
                    comp = phases[2 * (r ^ di) + (c ^ dj)]
                    if r & di:
                        comp = _shift_rows(comp)
                    if c & dj:
                        comp = _shift_cols(comp)
                    comps.append(comp)
            p = jnp.concatenate(comps, axis=-1).reshape(3136, 4 * HID)
            for s in (0, 1):
                for t in (0, 1):
                    y = jnp.dot(p, w_ref[2 * s + t],
                                preferred_element_type=jnp.float32)
                    y = jnp.maximum(y + b_ref[...], 0.0)
                    q2 = 4 * (2 * r + s) + (2 * c + t)
                    out_ref[0, :, :, q2 * D:(q2 + 1) * D] = (
                        y.reshape(56, 56, D))


def _dec3_body(y2_ref, wbig_ref, out_ref):
    y2 = y2_ref[0].reshape(3136, 16 * D)
    g = jnp.dot(y2, wbig_ref[...],
                preferred_element_type=jnp.float32)
    g = g.reshape(56, 56, 144)
    for u in range(4):
        for v in range(4):
            acc = jnp.zeros((56, 56), jnp.float32)
            for ki in range(3):
                for kj in range(3):
                    up, vp = u + 1 - ki, v + 1 - kj
                    ra, rb = up // 4, vp // 4
                    c = (4 * u + v) * 9 + 3 * ki + kj
                    plane = g[:, :, c]
                    if ra == 1:
                        plane = jnp.concatenate(
                            [plane[1:], jnp.zeros((1, 56), jnp.float32)], 0)
                    elif ra == -1:
                        plane = jnp.concatenate(
                            [jnp.zeros((1, 56), jnp.float32), plane[:-1]], 0)
                    if rb == 1:
                        plane = jnp.concatenate(
                            [plane[:, 1:], jnp.zeros((56, 1), jnp.float32)], 1)
                    elif rb == -1:
                        plane = jnp.concatenate(
                            [jnp.zeros((56, 1), jnp.float32), plane[:, :-1]], 1)
                    acc = acc + plane
            out_ref[0, 4 * u + v] = acc


def _decoder(quant_flat, dec_w1, dec_b1, dec_w2, dec_b2, dec_w3, dec_b3):
    x = quant_flat.reshape(B, 56, 56, D)
    w1 = _phase_w(dec_w1)
    w2 = _phase_w(dec_w2)
    wbig = jnp.zeros((16 * D, 144), dec_w3.dtype)
    for u in range(4):
        for v in range(4):
            for ki in range(3):
                for kj in range(3):
                    su, sv = (u + 1 - ki) % 4, (v + 1 - kj) % 4
                    q2 = 4 * su + sv
                    col = (4 * u + v) * 9 + 3 * ki + kj
                    wbig = wbig.at[q2 * D:(q2 + 1) * D, col].set(
                        dec_w3[:, 0, ki, kj])

    y1 = pl.pallas_call(
        _dec1_body,
        grid=(B,),
        in_specs=[
            pl.BlockSpec((1, 56, 56, D), lambda i: (i, 0, 0, 0)),
            pl.BlockSpec((4, 4 * D, HID), lambda i: (0, 0, 0)),
            pl.BlockSpec((1, HID), lambda i: (0, 0)),
        ],
        out_specs=pl.BlockSpec((1, 56, 56, 4 * HID), lambda i: (i, 0, 0, 0)),
        out_shape=jax.ShapeDtypeStruct((B, 56, 56, 4 * HID), jnp.float32),
    )(x, w1, dec_b1.reshape(1, HID))

    y2 = pl.pallas_call(
        _dec2_body,
        grid=(B,),
        in_specs=[
            pl.BlockSpec((1, 56, 56, 4 * HID), lambda i: (i, 0, 0, 0)),
            pl.BlockSpec((4, 4 * HID, D), lambda i: (0, 0, 0)),
            pl.BlockSpec((1, D), lambda i: (0, 0)),
        ],
        out_specs=pl.BlockSpec((1, 56, 56, 16 * D), lambda i: (i, 0, 0, 0)),
        out_shape=jax.ShapeDtypeStruct((B, 56, 56, 16 * D), jnp.float32),
    )(y1, w2, dec_b2.reshape(1, D))

    y3 = pl.pallas_call(
        _dec3_body,
        grid=(B,),
        in_specs=[
            pl.BlockSpec((1, 56, 56, 16 * D), lambda i: (i, 0, 0, 0)),
            pl.BlockSpec((16 * D, 144), lambda i: (0, 0)),
        ],
        out_specs=pl.BlockSpec((1, 16, 56, 56), lambda i: (i, 0, 0, 0)),
        out_shape=jax.ShapeDtypeStruct((B, 16, 56, 56), jnp.float32),
    )(y2, wbig)

    x_recon = (y3.reshape(B, 4, 4, 56, 56)
               .transpose(0, 3, 1, 4, 2)
               .reshape(B, 1, 224, 224)) + dec_b3[0]
    return x_recon


def _conv2d(x, w, b, stride, pad):
    y = jax.lax.conv_general_dilated(
        x, w, (stride, stride), ((pad, pad), (pad, pad)),
        dimension_numbers=('NCHW', 'OIHW', 'NCHW'))
    return y + b[None, :, None, None]


@jax.jit
def kernel(x, enc_w1, enc_b1, enc_w2, enc_b2, enc_w3, enc_b3, embeddings,
           dec_w1, dec_b1, dec_w2, dec_b2, dec_w3, dec_b3):
    h = jax.nn.relu(_conv2d(x, enc_w1, enc_b1, 2, 1))
    h = jax.nn.relu(_conv2d(h, enc_w2, enc_b2, 2, 1))
    z = _conv2d(h, enc_w3, enc_b3, 1, 1)
    z_e = jnp.transpose(z, (0, 2, 3, 1))
    flat = z_e.reshape(-1, D)

    quant_flat, counts, sqerr = _vq(flat, embeddings)

    loss = 1.25 * sqerr / (N_TOK * D)
    avg_probs = counts / N_TOK
    perplexity = jnp.exp(-jnp.sum(avg_probs * jnp.log(avg_probs + 1e-10)))

    x_recon = _decoder(quant_flat, dec_w1, dec_b1, dec_w2, dec_b2,
                       dec_w3, dec_b3)
    return (loss, x_recon, perplexity)


# speedup vs baseline: 3.5845x; 2.2300x over previous
import jax
import jax.numpy as jnp
from jax.experimental import pallas as pl

B, C_IN, H, W = 16, 1, 224, 224
HID = 128
K = 1024
D = 64
N_TOK = 16 * 56 * 56
TILE = 1024
GRID = N_TOK // TILE

_S = {0: ((0, 1),), 1: ((0, 2), (1, 0))}


def _vq_body(flat_ref, emb_ref, embt_ref, embsq_ref,
             quant_ref, counts_ref, sqerr_ref):
    i = pl.program_id(0)
    f = flat_ref[...]
    emb = emb_ref[...]
    scores = jax.lax.dot_general(
        f, emb, (((1,), (0,)), ((), ())),
        preferred_element_type=jnp.float32)
    row_sq = jnp.sum(f * f, axis=1, keepdims=True)
    d = (row_sq + embsq_ref[...]) - 2.0 * scores
    m = jnp.min(d, axis=1, keepdims=True)
    iota = jax.lax.broadcasted_iota(jnp.int32, (TILE, K), 1)
    idx = jnp.min(jnp.where(d == m, iota, K), axis=1)
    onehot = (iota == idx[:, None]).astype(jnp.float32)
    q = jax.lax.dot_general(
        onehot, embt_ref[...], (((1,), (0,)), ((), ())),
        preferred_element_type=jnp.float32,
        precision=jax.lax.Precision.HIGHEST)
    quant_ref[...] = q
    dsq = jnp.sum((q - f) ** 2)
    cnt = jnp.sum(onehot, axis=0, keepdims=True)

    @pl.when(i == 0)
    def _init():
        counts_ref[...] = cnt
        sqerr_ref[...] = jnp.full((1, 128), dsq, jnp.float32)

    @pl.when(i != 0)
    def _acc():
        counts_ref[...] += cnt
        sqerr_ref[...] += dsq


def _vq(flat, embeddings):
    emb_sq = jnp.sum(embeddings * embeddings, axis=0, keepdims=True)
    quant, counts, sqerr = pl.pallas_call(
        _vq_body,
        grid=(GRID,),
        in_specs=[
            pl.BlockSpec((TILE, D), lambda i: (i, 0)),
            pl.BlockSpec((D, K), lambda i: (0, 0)),
            pl.BlockSpec((K, D), lambda i: (0, 0)),
            pl.BlockSpec((1, K), lambda i: (0, 0)),
        ],
        out_specs=[
            pl.BlockSpec((TILE, D), lambda i: (i, 0)),
            pl.BlockSpec((1, K), lambda i: (0, 0)),
            pl.BlockSpec((1, 128), lambda i: (0, 0)),
        ],
        out_shape=[
            jax.ShapeDtypeStruct((N_TOK, D), jnp.float32),
            jax.ShapeDtypeStruct((1, K), jnp.float32),
            jax.ShapeDtypeStruct((1, 128), jnp.float32),
        ],
    )(flat, embeddings, embeddings.T, emb_sq)
    return quant, counts[0], sqerr[0, 0]


_ORDER = ((0, 1), (0, 0), (1, 0), (1, 1))
_PSLICE = ((1, 2), (0, 2), (1, 3), (0, 4))
_TAP = {0: {0: 1}, 1: {0: 2, 1: 0}}


def _exact_w(w):
    mats = []
    for r in (0, 1):
        for c in (0, 1):
            blocks = []
            for (di, dj) in _ORDER:
                if (r == 1 or di == 0) and (c == 1 or dj == 0):
                    blocks.append(w[:, :, _TAP[r][di], _TAP[c][dj]])
            mats.append(jnp.concatenate(blocks, axis=0))
    return mats


_L3_GROUPS = []
_L3_COLS = []
for _ra in (-1, 0, 1):
    for _rb in (-1, 0, 1):
        _start = len(_L3_COLS)
        for _u in range(4):
            for _v in range(4):
                for _ki in range(3):
                    for _kj in range(3):
                        if ((_u + 1 - _ki) // 4 == _ra
                                and (_v + 1 - _kj) // 4 == _rb):
                            _L3_COLS.append(
                                (4 * ((_u + 1 - _ki) % 4)
                                 + ((_v + 1 - _kj) % 4),
                                 3 * _ki + _kj, 4 * _u + _v))
        _L3_GROUPS.append((_ra, _rb, _start, len(_L3_COLS) - _start))


def _shift_rows(a):
    return jnp.concatenate(
        [a[1:], jnp.zeros((1,) + a.shape[1:], a.dtype)], axis=0)


def _shift_cols(a):
    return jnp.concatenate(
        [a[:, 1:], jnp.zeros((a.shape[0], 1) + a.shape[2:], a.dtype)], axis=1)


def _dec1_body(x_ref, w0_ref, w1_ref, w2_ref, w3_ref, b_ref, out_ref):
    xb = x_ref[0]
    comps = []
    for (di, dj) in _ORDER:
        comp = xb
        if di:
            comp = _shift_rows(comp)
        if dj:
            comp = _shift_cols(comp)
        comps.append(comp)
    p = jnp.concatenate(comps, axis=-1).reshape(3136, 4 * D)
    wrefs = (w0_ref, w1_ref, w2_ref, w3_ref)
    for q in range(4):
        lo, hi = _PSLICE[q]
        y = jnp.dot(p[:, lo * D:hi * D], wrefs[q][...],
                    preferred_element_type=jnp.float32)
        y = jnp.maximum(y + b_ref[...], 0.0)
        out_ref[0, :, :, q * HID:(q + 1) * HID] = y.reshape(56, 56, HID)


def _dec2_body(y1_ref, w0_ref, w1_ref, w2_ref, w3_ref, b_ref, out_ref):
    y1 = y1_ref[0]
    phases = [y1[:, :, q * HID:(q + 1) * HID] for q in range(4)]
    wrefs = (w0_ref, w1_ref, w2_ref, w3_ref)
    for r in (0, 1):
        for c in (0, 1):
            comps = []
            for (di, dj) in _ORDER:
                comp = phases[2 * (r ^ di) + (c ^ dj)]
                if r & di:
                    comp = _shift_rows(comp)
                if c & dj:
                    comp = _shift_cols(comp)
                comps.append(comp)
            p = jnp.concatenate(comps, axis=-1).reshape(3136, 4 * HID)
            for s in (0, 1):
                for t in (0, 1):
                    lo, hi = _PSLICE[2 * s + t]
                    y = jnp.dot(p[:, lo * HID:hi * HID], wrefs[2 * s + t][...],
                                preferred_element_type=jnp.float32)
                    y = jnp.maximum(y + b_ref[...], 0.0)
                    q2 = 4 * (2 * r + s) + (2 * c + t)
                    out_ref[0, :, :, q2 * D:(q2 + 1) * D] = (
                        y.reshape(56, 56, D))


def _dec3_body(y2_ref, wbig_ref, red_ref, out_ref):
    y2 = y2_ref[0].reshape(3136, 16 * D)
    g = jnp.dot(y2, wbig_ref[...],
                preferred_element_type=jnp.float32)
    b_idx = jax.lax.broadcasted_iota(jnp.int32, (3136, 1), 0) % 56
    parts = []
    for (ra, rb, start, size) in _L3_GROUPS:
        blk = g[:, start:start + size]
        if ra == 1:
            blk = jnp.concatenate(
                [blk[56:], jnp.zeros((56, size), jnp.float32)], 0)
        elif ra == -1:
            blk = jnp.concatenate(
                [jnp.zeros((56, size), jnp.float32), blk[:-56]], 0)
        if rb == 1:
            blk = jnp.concatenate(
                [blk[1:], jnp.zeros((1, size), jnp.float32)], 0)
            blk = jnp.where(b_idx == 55, 0.0, blk)
        elif rb == -1:
            blk = jnp.concatenate(
                [jnp.zeros((1, size), jnp.float32), blk[:-1]], 0)
            blk = jnp.where(b_idx == 0, 0.0, blk)
        parts.append(blk)
    gs = jnp.concatenate(parts, axis=1)
    y = jnp.dot(gs, red_ref[...],
                preferred_element_type=jnp.float32)
    out_ref[0] = y.reshape(56, 56, 16)


import numpy as _np

_L3_SEL = _np.zeros((144, 9), _np.float32)
_L3_BLK = _np.zeros((144, 16), _np.float32)
_L3_RED = _np.zeros((144, 16), _np.float32)
for _c, (_q2, _tap, _qo) in enumerate(_L3_COLS):
    _L3_SEL[_c, _tap] = 1.0
    _L3_BLK[_c, _q2] = 1.0
    _L3_RED[_c, _qo] = 1.0


def _decoder(quant_flat, dec_w1, dec_b1, dec_w2, dec_b2, dec_w3, dec_b3):
    x = quant_flat.reshape(B, 56, 56, D)
    w1 = _exact_w(dec_w1)
    w2 = _exact_w(dec_w2)
    tapw = dec_w3[:, 0].reshape(D, 9) @ jnp.asarray(_L3_SEL.T)
    wbig = (jnp.asarray(_L3_BLK.T)[:, None, :] * tapw[None]).reshape(
        16 * D, 144)
    red = jnp.asarray(_L3_RED)

    full = lambda *s: [pl.BlockSpec(m.shape, lambda i: (0,) * m.ndim)
                       for m in s]

    y1 = pl.pallas_call(
        _dec1_body,
        grid=(B,),
        in_specs=[pl.BlockSpec((1, 56, 56, D), lambda i: (i, 0, 0, 0))]
        + full(*w1) + full(dec_b1.reshape(1, HID)),
        out_specs=pl.BlockSpec((1, 56, 56, 4 * HID), lambda i: (i, 0, 0, 0)),
        out_shape=jax.ShapeDtypeStruct((B, 56, 56, 4 * HID), jnp.float32),
    )(x, *w1, dec_b1.reshape(1, HID))

    y2 = pl.pallas_call(
        _dec2_body,
        grid=(B,),
        in_specs=[pl.BlockSpec((1, 56, 56, 4 * HID), lambda i: (i, 0, 0, 0))]
        + full(*w2) + full(dec_b2.reshape(1, D)),
        out_specs=pl.BlockSpec((1, 56, 56, 16 * D), lambda i: (i, 0, 0, 0)),
        out_shape=jax.ShapeDtypeStruct((B, 56, 56, 16 * D), jnp.float32),
    )(y1, *w2, dec_b2.reshape(1, D))

    y3 = pl.pallas_call(
        _dec3_body,
        grid=(B,),
        in_specs=[pl.BlockSpec((1, 56, 56, 16 * D), lambda i: (i, 0, 0, 0))]
        + full(wbig, red),
        out_specs=pl.BlockSpec((1, 56, 56, 16), lambda i: (i, 0, 0, 0)),
        out_shape=jax.ShapeDtypeStruct((B, 56, 56, 16), jnp.float32),
    )(y2, wbig, red)

    x_recon = (y3.reshape(B, 56, 56, 4, 4)
               .transpose(0, 1, 3, 2, 4)
               .reshape(B, 1, 224, 224)) + dec_b3[0]
    return x_recon


def _conv2d(x, w, b, stride, pad):
    y = jax.lax.conv_general_dilated(
        x, w, (stride, stride), ((pad, pad), (pad, pad)),
        dimension_numbers=('NCHW', 'OIHW', 'NCHW'))
    return y + b[None, :, None, None]


@jax.jit
def kernel(x, enc_w1, enc_b1, enc_w2, enc_b2, enc_w3, enc_b3, embeddings,
           dec_w1, dec_b1, dec_w2, dec_b2, dec_w3, dec_b3):
    h = jax.nn.relu(_conv2d(x, enc_w1, enc_b1, 2, 1))
    h = jax.nn.relu(_conv2d(h, enc_w2, enc_b2, 2, 1))
    z = _conv2d(h, enc_w3, enc_b3, 1, 1)
    z_e = jnp.transpose(z, (0, 2, 3, 1))
    flat = z_e.reshape(-1, D)

    quant_flat, counts, sqerr = _vq(flat, embeddings)

    loss = 1.25 * sqerr / (N_TOK * D)
    avg_probs = counts / N_TOK
    perplexity = jnp.exp(-jnp.sum(avg_probs * jnp.log(avg_probs + 1e-10)))

    x_recon = _decoder(quant_flat, dec_w1, dec_b1, dec_w2, dec_b2,
                       dec_w3, dec_b3)
    return (loss, x_recon, perplexity)


# speedup vs baseline: 4.2376x; 1.1822x over previous
import jax
import jax.numpy as jnp
from jax.experimental import pallas as pl

B, C_IN, H, W = 16, 1, 224, 224
HID = 128
K = 1024
D = 64
N_TOK = 16 * 56 * 56
TILE = 1024
GRID = N_TOK // TILE

_S = {0: ((0, 1),), 1: ((0, 2), (1, 0))}


def _vq_body(flat_ref, emb_ref, embt_ref, embsq_ref,
             quant_ref, counts_ref, sqerr_ref):
    i = pl.program_id(0)
    f = flat_ref[...]
    emb = emb_ref[...]
    scores = jax.lax.dot_general(
        f, emb, (((1,), (0,)), ((), ())),
        preferred_element_type=jnp.float32)
    row_sq = jnp.sum(f * f, axis=1, keepdims=True)
    d = (row_sq + embsq_ref[...]) - 2.0 * scores
    m = jnp.min(d, axis=1, keepdims=True)
    iota = jax.lax.broadcasted_iota(jnp.int32, (TILE, K), 1)
    idx = jnp.min(jnp.where(d == m, iota, K), axis=1)
    onehot = (iota == idx[:, None]).astype(jnp.float32)
    q = jax.lax.dot_general(
        onehot, embt_ref[...], (((1,), (0,)), ((), ())),
        preferred_element_type=jnp.float32)
    quant_ref[...] = q
    dsq = jnp.sum((q - f) ** 2)
    cnt = jnp.sum(onehot, axis=0, keepdims=True)

    @pl.when(i == 0)
    def _init():
        counts_ref[...] = cnt
        sqerr_ref[...] = jnp.full((1, 128), dsq, jnp.float32)

    @pl.when(i != 0)
    def _acc():
        counts_ref[...] += cnt
        sqerr_ref[...] += dsq


def _vq(flat, embeddings):
    emb_sq = jnp.sum(embeddings * embeddings, axis=0, keepdims=True)
    quant, counts, sqerr = pl.pallas_call(
        _vq_body,
        grid=(GRID,),
        in_specs=[
            pl.BlockSpec((TILE, D), lambda i: (i, 0)),
            pl.BlockSpec((D, K), lambda i: (0, 0)),
            pl.BlockSpec((K, D), lambda i: (0, 0)),
            pl.BlockSpec((1, K), lambda i: (0, 0)),
        ],
        out_specs=[
            pl.BlockSpec((TILE, D), lambda i: (i, 0)),
            pl.BlockSpec((1, K), lambda i: (0, 0)),
            pl.BlockSpec((1, 128), lambda i: (0, 0)),
        ],
        out_shape=[
            jax.ShapeDtypeStruct((N_TOK, D), jnp.float32),
            jax.ShapeDtypeStruct((1, K), jnp.float32),
            jax.ShapeDtypeStruct((1, 128), jnp.float32),
        ],
    )(flat, embeddings, embeddings.T, emb_sq)
    return quant, counts[0], sqerr[0, 0]


_ORDER = ((0, 1), (0, 0), (1, 0), (1, 1))
_PSLICE = ((1, 2), (0, 2), (1, 3), (0, 4))
_TAP = {0: {0: 1}, 1: {0: 2, 1: 0}}


def _exact_w(w):
    mats = []
    for r in (0, 1):
        for c in (0, 1):
            blocks = []
            for (di, dj) in _ORDER:
                if (r == 1 or di == 0) and (c == 1 or dj == 0):
                    blocks.append(w[:, :, _TAP[r][di], _TAP[c][dj]])
            mats.append(jnp.concatenate(blocks, axis=0))
    return mats


_L3_GROUPS = []
_L3_COLS = []
for _ra in (-1, 0, 1):
    for _rb in (-1, 0, 1):
        _start = len(_L3_COLS)
        for _u in range(4):
            for _v in range(4):
                for _ki in range(3):
                    for _kj in range(3):
                        if ((_u + 1 - _ki) // 4 == _ra
                                and (_v + 1 - _kj) // 4 == _rb):
                            _L3_COLS.append(
                                (4 * ((_u + 1 - _ki) % 4)
                                 + ((_v + 1 - _kj) % 4),
                                 3 * _ki + _kj, 4 * _u + _v))
        _L3_GROUPS.append((_ra, _rb, _start, len(_L3_COLS) - _start))


def _shift_rows(a):
    return jnp.concatenate(
        [a[1:], jnp.zeros((1,) + a.shape[1:], a.dtype)], axis=0)


def _shift_cols(a):
    return jnp.concatenate(
        [a[:, 1:], jnp.zeros((a.shape[0], 1) + a.shape[2:], a.dtype)], axis=1)


def _dec1_body(x_ref, w0_ref, w1_ref, w2_ref, w3_ref, b_ref, out_ref):
    xb = x_ref[0]
    comps = []
    for (di, dj) in _ORDER:
        comp = xb
        if di:
            comp = _shift_rows(comp)
        if dj:
            comp = _shift_cols(comp)
        comps.append(comp)
    p = jnp.concatenate(comps, axis=-1).reshape(3136, 4 * D)
    wrefs = (w0_ref, w1_ref, w2_ref, w3_ref)
    for q in range(4):
        lo, hi = _PSLICE[q]
        y = jnp.dot(p[:, lo * D:hi * D], wrefs[q][...],
                    preferred_element_type=jnp.float32)
        y = jnp.maximum(y + b_ref[...], 0.0)
        out_ref[0, :, :, q * HID:(q + 1) * HID] = y.reshape(56, 56, HID)


def _dec2_body(y1_ref, w0_ref, w1_ref, w2_ref, w3_ref, b_ref, out_ref):
    y1 = y1_ref[0]
    phases = [y1[:, :, q * HID:(q + 1) * HID] for q in range(4)]
    wrefs = (w0_ref, w1_ref, w2_ref, w3_ref)
    for r in (0, 1):
        for c in (0, 1):
            comps = []
            for (di, dj) in _ORDER:
                comp = phases[2 * (r ^ di) + (c ^ dj)]
                if r & di:
                    comp = _shift_rows(comp)
                if c & dj:
                    comp = _shift_cols(comp)
                comps.append(comp)
            p = jnp.concatenate(comps, axis=-1).reshape(3136, 4 * HID)
            for s in (0, 1):
                for t in (0, 1):
                    lo, hi = _PSLICE[2 * s + t]
                    y = jnp.dot(p[:, lo * HID:hi * HID], wrefs[2 * s + t][...],
                                preferred_element_type=jnp.float32)
                    y = jnp.maximum(y + b_ref[...], 0.0)
                    q2 = 4 * (2 * r + s) + (2 * c + t)
                    out_ref[0, :, :, q2 * D:(q2 + 1) * D] = (
                        y.reshape(56, 56, D))


def _dec3_body(y2_ref, wbig_ref, red_ref, out_ref):
    y2 = y2_ref[0].reshape(3136, 16 * D)
    g = jnp.dot(y2, wbig_ref[...],
                preferred_element_type=jnp.float32)
    b_idx = jax.lax.broadcasted_iota(jnp.int32, (3136, 1), 0) % 56
    parts = []
    for (ra, rb, start, size) in _L3_GROUPS:
        blk = g[:, start:start + size]
        if ra == 1:
            blk = jnp.concatenate(
                [blk[56:], jnp.zeros((56, size), jnp.float32)], 0)
        elif ra == -1:
            blk = jnp.concatenate(
                [jnp.zeros((56, size), jnp.float32), blk[:-56]], 0)
        if rb == 1:
            blk = jnp.concatenate(
                [blk[1:], jnp.zeros((1, size), jnp.float32)], 0)
            blk = jnp.where(b_idx == 55, 0.0, blk)
        elif rb == -1:
            blk = jnp.concatenate(
                [jnp.zeros((1, size), jnp.float32), blk[:-1]], 0)
            blk = jnp.where(b_idx == 0, 0.0, blk)
        parts.append(blk)
    gs = jnp.concatenate(parts, axis=1)
    y = jnp.dot(gs, red_ref[...],
                preferred_element_type=jnp.float32)
    out_ref[0] = y.reshape(56, 56, 16)


import numpy as _np

_L3_SEL = _np.zeros((144, 9), _np.float32)
_L3_BLK = _np.zeros((144, 16), _np.float32)
_L3_RED = _np.zeros((144, 16), _np.float32)
for _c, (_q2, _tap, _qo) in enumerate(_L3_COLS):
    _L3_SEL[_c, _tap] = 1.0
    _L3_BLK[_c, _q2] = 1.0
    _L3_RED[_c, _qo] = 1.0


def _decoder(quant_flat, dec_w1, dec_b1, dec_w2, dec_b2, dec_w3, dec_b3):
    x = quant_flat.reshape(B, 56, 56, D)
    w1 = _exact_w(dec_w1)
    w2 = _exact_w(dec_w2)
    tapw = dec_w3[:, 0].reshape(D, 9) @ jnp.asarray(_L3_SEL.T)
    wbig = (jnp.asarray(_L3_BLK.T)[:, None, :] * tapw[None]).reshape(
        16 * D, 144)
    red = jnp.asarray(_L3_RED)

    full = lambda *s: [pl.BlockSpec(m.shape, lambda i: (0,) * m.ndim)
                       for m in s]

    y1 = pl.pallas_call(
        _dec1_body,
        grid=(B,),
        in_specs=[pl.BlockSpec((1, 56, 56, D), lambda i: (i, 0, 0, 0))]
        + full(*w1) + full(dec_b1.reshape(1, HID)),
        out_specs=pl.BlockSpec((1, 56, 56, 4 * HID), lambda i: (i, 0, 0, 0)),
        out_shape=jax.ShapeDtypeStruct((B, 56, 56, 4 * HID), jnp.float32),
    )(x, *w1, dec_b1.reshape(1, HID))

    y2 = pl.pallas_call(
        _dec2_body,
        grid=(B,),
        in_specs=[pl.BlockSpec((1, 56, 56, 4 * HID), lambda i: (i, 0, 0, 0))]
        + full(*w2) + full(dec_b2.reshape(1, D)),
        out_specs=pl.BlockSpec((1, 56, 56, 16 * D), lambda i: (i, 0, 0, 0)),
        out_shape=jax.ShapeDtypeStruct((B, 56, 56, 16 * D), jnp.float32),
    )(y1, *w2, dec_b2.reshape(1, D))

    y3 = pl.pallas_call(
        _dec3_body,
        grid=(B,),
        in_specs=[pl.BlockSpec((1, 56, 56, 16 * D), lambda i: (i, 0, 0, 0))]
        + full(wbig, red),
        out_specs=pl.BlockSpec((1, 56, 56, 16), lambda i: (i, 0, 0, 0)),
        out_shape=jax.ShapeDtypeStruct((B, 56, 56, 16), jnp.float32),
    )(y2, wbig, red)

    x_recon = (y3.reshape(B, 56, 56, 4, 4)
               .transpose(0, 1, 3, 2, 4)
               .reshape(B, 1, 224, 224)) + dec_b3[0]
    return x_recon


def _conv2d(x, w, b, stride, pad):
    y = jax.lax.conv_general_dilated(
        x, w, (stride, stride), ((pad, pad), (pad, pad)),
        dimension_numbers=('NCHW', 'OIHW', 'NCHW'))
    return y + b[None, :, None, None]


@jax.jit
def kernel(x, enc_w1, enc_b1, enc_w2, enc_b2, enc_w3, enc_b3, embeddings,
           dec_w1, dec_b1, dec_w2, dec_b2, dec_w3, dec_b3):
    h = jax.nn.relu(_conv2d(x, enc_w1, enc_b1, 2, 1))
    h = jax.nn.relu(_conv2d(h, enc_w2, enc_b2, 2, 1))
    z = _conv2d(h, enc_w3, enc_b3, 1, 1)
    z_e = jnp.transpose(z, (0, 2, 3, 1))
    flat = z_e.reshape(-1, D)

    quant_flat, counts, sqerr = _vq(flat, embeddings)

    loss = 1.25 * sqerr / (N_TOK * D)
    avg_probs = counts / N_TOK
    perplexity = jnp.exp(-jnp.sum(avg_probs * jnp.log(avg_probs + 1e-10)))

    x_recon = _decoder(quant_flat, dec_w1, dec_b1, dec_w2, dec_b2,
                       dec_w3, dec_b3)
    return (loss, x_recon, perplexity)


# speedup vs baseline: 4.3126x; 1.0177x over previous
import jax
import jax.numpy as jnp
from jax.experimental import pallas as pl

B, C_IN, H, W = 16, 1, 224, 224
HID = 128
K = 1024
D = 64
N_TOK = 16 * 56 * 56
TILE = 1792
GRID = N_TOK // TILE

_S = {0: ((0, 1),), 1: ((0, 2), (1, 0))}


def _vq_body(flat_ref, emb_ref, embt_ref, embsq_ref,
             quant_ref, counts_ref, sqerr_ref):
    i = pl.program_id(0)
    f = flat_ref[...]
    emb = emb_ref[...]
    scores = jax.lax.dot_general(
        f, emb, (((1,), (0,)), ((), ())),
        preferred_element_type=jnp.float32)
    row_sq = jnp.sum(f * f, axis=1, keepdims=True)
    d = (row_sq + embsq_ref[...]) - 2.0 * scores
    m = jnp.min(d, axis=1, keepdims=True)
    iota = jax.lax.broadcasted_iota(jnp.int32, (TILE, K), 1)
    idx = jnp.min(jnp.where(d == m, iota, K), axis=1)
    onehot = (iota == idx[:, None]).astype(jnp.float32)
    q = jax.lax.dot_general(
        onehot, embt_ref[...], (((1,), (0,)), ((), ())),
        preferred_element_type=jnp.float32)
    quant_ref[...] = q
    dsq = jnp.sum((q - f) ** 2)
    cnt = jnp.sum(onehot, axis=0, keepdims=True)

    @pl.when(i == 0)
    def _init():
        counts_ref[...] = cnt
        sqerr_ref[...] = jnp.full((1, 128), dsq, jnp.float32)

    @pl.when(i != 0)
    def _acc():
        counts_ref[...] += cnt
        sqerr_ref[...] += dsq


def _vq(flat, embeddings):
    emb_sq = jnp.sum(embeddings * embeddings, axis=0, keepdims=True)
    quant, counts, sqerr = pl.pallas_call(
        _vq_body,
        grid=(GRID,),
        in_specs=[
            pl.BlockSpec((TILE, D), lambda i: (i, 0)),
            pl.BlockSpec((D, K), lambda i: (0, 0)),
            pl.BlockSpec((K, D), lambda i: (0, 0)),
            pl.BlockSpec((1, K), lambda i: (0, 0)),
        ],
        out_specs=[
            pl.BlockSpec((TILE, D), lambda i: (i, 0)),
            pl.BlockSpec((1, K), lambda i: (0, 0)),
            pl.BlockSpec((1, 128), lambda i: (0, 0)),
        ],
        out_shape=[
            jax.ShapeDtypeStruct((N_TOK, D), jnp.float32),
            jax.ShapeDtypeStruct((1, K), jnp.float32),
            jax.ShapeDtypeStruct((1, 128), jnp.float32),
        ],
    )(flat, embeddings, embeddings.T, emb_sq)
    return quant, counts[0], sqerr[0, 0]


_ORDER = ((0, 1), (0, 0), (1, 0), (1, 1))
_PSLICE = ((1, 2), (0, 2), (1, 3), (0, 4))
_TAP = {0: {0: 1}, 1: {0: 2, 1: 0}}


def _exact_w(w):
    mats = []
    for r in (0, 1):
        for c in (0, 1):
            blocks = []
            for (di, dj) in _ORDER:
                if (r == 1 or di == 0) and (c == 1 or dj == 0):
                    blocks.append(w[:, :, _TAP[r][di], _TAP[c][dj]])
            mats.append(jnp.concatenate(blocks, axis=0))
    return mats


_L3_GROUPS = []
_L3_COLS = []
for _ra in (-1, 0, 1):
    for _rb in (-1, 0, 1):
        _start = len(_L3_COLS)
        for _u in range(4):
            for _v in range(4):
                for _ki in range(3):
                    for _kj in range(3):
                        if ((_u + 1 - _ki) // 4 == _ra
                                and (_v + 1 - _kj) // 4 == _rb):
                            _L3_COLS.append(
                                (4 * ((_u + 1 - _ki) % 4)
                                 + ((_v + 1 - _kj) % 4),
                                 3 * _ki + _kj, 4 * _u + _v))
        _L3_GROUPS.append((_ra, _rb, _start, len(_L3_COLS) - _start))


def _shift_rows(a):
    return jnp.concatenate(
        [a[1:], jnp.zeros((1,) + a.shape[1:], a.dtype)], axis=0)


def _shift_cols(a):
    return jnp.concatenate(
        [a[:, 1:], jnp.zeros((a.shape[0], 1) + a.shape[2:], a.dtype)], axis=1)


def _dec1_body(x_ref, w0_ref, w1_ref, w2_ref, w3_ref, b_ref, out_ref):
    xb = x_ref[0]
    comps = []
    for (di, dj) in _ORDER:
        comp = xb
        if di:
            comp = _shift_rows(comp)
        if dj:
            comp = _shift_cols(comp)
        comps.append(comp)
    p = jnp.concatenate(comps, axis=-1).reshape(3136, 4 * D)
    wrefs = (w0_ref, w1_ref, w2_ref, w3_ref)
    for q in range(4):
        lo, hi = _PSLICE[q]
        y = jnp.dot(p[:, lo * D:hi * D], wrefs[q][...],
                    preferred_element_type=jnp.float32)
        y = jnp.maximum(y + b_ref[...], 0.0)
        out_ref[0, :, :, q * HID:(q + 1) * HID] = y.reshape(56, 56, HID)


def _dec2_body(y1_ref, w0_ref, w1_ref, w2_ref, w3_ref, b_ref, out_ref):
    y1 = y1_ref[0]
    phases = [y1[:, :, q * HID:(q + 1) * HID] for q in range(4)]
    wrefs = (w0_ref, w1_ref, w2_ref, w3_ref)
    for r in (0, 1):
        for c in (0, 1):
            comps = []
            for (di, dj) in _ORDER:
                comp = phases[2 * (r ^ di) + (c ^ dj)]
                if r & di:
                    comp = _shift_rows(comp)
                if c & dj:
                    comp = _shift_cols(comp)
                comps.append(comp)
            p = jnp.concatenate(comps, axis=-1).reshape(3136, 4 * HID)
            for s in (0, 1):
                for t in (0, 1):
                    lo, hi = _PSLICE[2 * s + t]
                    y = jnp.dot(p[:, lo * HID:hi * HID], wrefs[2 * s + t][...],
                                preferred_element_type=jnp.float32)
                    y = jnp.maximum(y + b_ref[...], 0.0)
                    q2 = 4 * (2 * r + s) + (2 * c + t)
                    out_ref[0, :, :, q2 * D:(q2 + 1) * D] = (
                        y.reshape(56, 56, D))


def _dec3_body(y2_ref, wbig_ref, red_ref, out_ref):
    y2 = y2_ref[0].reshape(3136, 16 * D)
    g = jnp.dot(y2, wbig_ref[...],
                preferred_element_type=jnp.float32)
    b_idx = jax.lax.broadcasted_iota(jnp.int32, (3136, 1), 0) % 56
    parts = []
    for (ra, rb, start, size) in _L3_GROUPS:
        blk = g[:, start:start + size]
        if ra == 1:
            blk = jnp.concatenate(
                [blk[56:], jnp.zeros((56, size), jnp.float32)], 0)
        elif ra == -1:
            blk = jnp.concatenate(
                [jnp.zeros((56, size), jnp.float32), blk[:-56]], 0)
        if rb == 1:
            blk = jnp.concatenate(
                [blk[1:], jnp.zeros((1, size), jnp.float32)], 0)
            blk = jnp.where(b_idx == 55, 0.0, blk)
        elif rb == -1:
            blk = jnp.concatenate(
                [jnp.zeros((1, size), jnp.float32), blk[:-1]], 0)
            blk = jnp.where(b_idx == 0, 0.0, blk)
        parts.append(blk)
    gs = jnp.concatenate(parts, axis=1)
    y = jnp.dot(gs, red_ref[...],
                preferred_element_type=jnp.float32)
    out_ref[0] = y.reshape(56, 56, 16)


import numpy as _np

_L3_SEL = _np.zeros((144, 9), _np.float32)
_L3_BLK = _np.zeros((144, 16), _np.float32)
_L3_RED = _np.zeros((144, 16), _np.float32)
for _c, (_q2, _tap, _qo) in enumerate(_L3_COLS):
    _L3_SEL[_c, _tap] = 1.0
    _L3_BLK[_c, _q2] = 1.0
    _L3_RED[_c, _qo] = 1.0


def _decoder(quant_flat, dec_w1, dec_b1, dec_w2, dec_b2, dec_w3, dec_b3):
    x = quant_flat.reshape(B, 56, 56, D)
    w1 = _exact_w(dec_w1)
    w2 = _exact_w(dec_w2)
    tapw = dec_w3[:, 0].reshape(D, 9) @ jnp.asarray(_L3_SEL.T)
    wbig = (jnp.asarray(_L3_BLK.T)[:, None, :] * tapw[None]).reshape(
        16 * D, 144)
    red = jnp.asarray(_L3_RED)

    full = lambda *s: [pl.BlockSpec(m.shape, lambda i: (0,) * m.ndim)
                       for m in s]

    y1 = pl.pallas_call(
        _dec1_body,
        grid=(B,),
        in_specs=[pl.BlockSpec((1, 56, 56, D), lambda i: (i, 0, 0, 0))]
        + full(*w1) + full(dec_b1.reshape(1, HID)),
        out_specs=pl.BlockSpec((1, 56, 56, 4 * HID), lambda i: (i, 0, 0, 0)),
        out_shape=jax.ShapeDtypeStruct((B, 56, 56, 4 * HID), jnp.float32),
    )(x, *w1, dec_b1.reshape(1, HID))

    y2 = pl.pallas_call(
        _dec2_body,
        grid=(B,),
        in_specs=[pl.BlockSpec((1, 56, 56, 4 * HID), lambda i: (i, 0, 0, 0))]
        + full(*w2) + full(dec_b2.reshape(1, D)),
        out_specs=pl.BlockSpec((1, 56, 56, 16 * D), lambda i: (i, 0, 0, 0)),
        out_shape=jax.ShapeDtypeStruct((B, 56, 56, 16 * D), jnp.float32),
    )(y1, *w2, dec_b2.reshape(1, D))

    y3 = pl.pallas_call(
        _dec3_body,
        grid=(B,),
        in_specs=[pl.BlockSpec((1, 56, 56, 16 * D), lambda i: (i, 0, 0, 0))]
        + full(wbig, red),
        out_specs=pl.BlockSpec((1, 56, 56, 16), lambda i: (i, 0, 0, 0)),
        out_shape=jax.ShapeDtypeStruct((B, 56, 56, 16), jnp.float32),
    )(y2, wbig, red)

    x_recon = (y3.reshape(B, 56, 56, 4, 4)
               .transpose(0, 1, 3, 2, 4)
               .reshape(B, 1, 224, 224)) + dec_b3[0]
    return x_recon


def _conv2d(x, w, b, stride, pad):
    y = jax.lax.conv_general_dilated(
        x, w, (stride, stride), ((pad, pad), (pad, pad)),
        dimension_numbers=('NCHW', 'OIHW', 'NCHW'))
    return y + b[None, :, None, None]


@jax.jit
def kernel(x, enc_w1, enc_b1, enc_w2, enc_b2, enc_w3, enc_b3, embeddings,
           dec_w1, dec_b1, dec_w2, dec_b2, dec_w3, dec_b3):
    h = jax.nn.relu(_conv2d(x, enc_w1, enc_b1, 2, 1))
    h = jax.nn.relu(_conv2d(h, enc_w2, enc_b2, 2, 1))
    z = _conv2d(h, enc_w3, enc_b3, 1, 1)
    z_e = jnp.transpose(z, (0, 2, 3, 1))
    flat = z_e.reshape(-1, D)

    quant_flat, counts, sqerr = _vq(flat, embeddings)

    loss = 1.25 * sqerr / (N_TOK * D)
    avg_probs = counts / N_TOK
    perplexity = jnp.exp(-jnp.sum(avg_probs * jnp.log(avg_probs + 1e-10)))

    x_recon = _decoder(quant_flat, dec_w1, dec_b1, dec_w2, dec_b2,
                       dec_w3, dec_b3)
    return (loss, x_recon, perplexity)
